# baseline (speedup 1.0000x reference)
import jax
import jax.numpy as jnp
from jax.experimental import pallas as pl

CLS_NUM = 1000
D = 512
N_PREFIX = 3
N_SUFFIX = 2
N_CTX = N_PREFIX + N_SUFFIX
N_CLS_TOK = 3
CTX_LEN = 77
PAD_SIZE = CTX_LEN - (N_CTX + N_CLS_TOK + 2)
PAD_LEN = 75

BLOCK_C = 40


def _build_kernel(ctx_ref, sot_ref, cls_ref, eot_ref, pad_ref, out_ref):
    b = out_ref.shape[0]
    out_ref[:, 0:1, :] = jnp.broadcast_to(sot_ref[0, :, :][None], (b, 1, D))
    out_ref[:, 1 : 1 + N_PREFIX, :] = jnp.broadcast_to(
        ctx_ref[0, :N_PREFIX, :][None], (b, N_PREFIX, D)
    )
    out_ref[:, 1 + N_PREFIX : 1 + N_PREFIX + N_CLS_TOK, :] = cls_ref[...]
    out_ref[:, 1 + N_PREFIX + N_CLS_TOK : 1 + N_CTX + N_CLS_TOK, :] = (
        jnp.broadcast_to(ctx_ref[0, N_PREFIX:, :][None], (b, N_SUFFIX, D))
    )
    out_ref[:, 1 + N_CTX + N_CLS_TOK : 2 + N_CTX + N_CLS_TOK, :] = (
        jnp.broadcast_to(eot_ref[0, :, :][None], (b, 1, D))
    )
    out_ref[:, 2 + N_CTX + N_CLS_TOK :, :] = jnp.broadcast_to(
        pad_ref[0, :PAD_SIZE, :][None], (b, PAD_SIZE, D)
    )


def kernel(ctx, emb_sot, emb_cls, emb_eot, emb_pad):
    grid = (CLS_NUM // BLOCK_C,)
    return pl.pallas_call(
        _build_kernel,
        grid=grid,
        in_specs=[
            pl.BlockSpec((1, N_CTX, D), lambda i: (0, 0, 0)),
            pl.BlockSpec((1, 1, D), lambda i: (0, 0, 0)),
            pl.BlockSpec((BLOCK_C, N_CLS_TOK, D), lambda i: (i, 0, 0)),
            pl.BlockSpec((1, 1, D), lambda i: (0, 0, 0)),
            pl.BlockSpec((1, PAD_LEN, D), lambda i: (0, 0, 0)),
        ],
        out_specs=pl.BlockSpec((BLOCK_C, CTX_LEN, D), lambda i: (i, 0, 0)),
        out_shape=jax.ShapeDtypeStruct((CLS_NUM, CTX_LEN, D), jnp.float32),
    )(ctx, emb_sot, emb_cls, emb_eot, emb_pad)
